# baseline (speedup 1.0000x reference)
import jax
import jax.numpy as jnp
from jax.experimental import pallas as pl
from jax.experimental.pallas import tpu as pltpu

_B, _S, _D = 4, 8192, 2048
_ST, _EN = 0, 1024
_W = _EN - _ST
_NCHUNK = 4
_CS = _S // _NCHUNK


def _body(eot_ref, w_ref, hid_ref, out_ref, rows_s, new_s, copy_sem, row_sem):
    bulk = []
    for b in range(_B):
        for k in range(_NCHUNK):
            cp = pltpu.make_async_copy(
                hid_ref.at[pl.ds(b, 1), pl.ds(k * _CS, _CS), :],
                out_ref.at[pl.ds(b, 1), pl.ds(k * _CS, _CS), :],
                copy_sem,
            )
            cp.start()
            bulk.append(cp)

    gathers = []
    for b in range(_B):
        e = eot_ref[b]
        cp = pltpu.make_async_copy(
            hid_ref.at[pl.ds(b, 1), pl.ds(e, 1), pl.ds(_ST, _W)],
            rows_s.at[pl.ds(b, 1)],
            row_sem,
        )
        cp.start()
        gathers.append(cp)
    for cp in gathers:
        cp.wait()

    t = rows_s[...].reshape(_B, _W)
    r = jax.lax.dot_general(
        t, w_ref[...], (((1,), (0,)), ((), ())),
        preferred_element_type=jnp.float32,
    )
    inv = jax.lax.dot_general(
        r, w_ref[...], (((1,), (1,)), ((), ())),
        preferred_element_type=jnp.float32,
    )
    new_s[...] = inv.reshape(_B, 1, _W)

    for cp in bulk:
        cp.wait()

    patches = []
    for b in range(_B):
        e = eot_ref[b]
        cp = pltpu.make_async_copy(
            new_s.at[pl.ds(b, 1)],
            out_ref.at[pl.ds(b, 1), pl.ds(e, 1), pl.ds(_ST, _W)],
            row_sem,
        )
        cp.start()
        patches.append(cp)
    for cp in patches:
        cp.wait()


def kernel(hidden_states, eot_indices, W):
    eot = eot_indices.astype(jnp.int32)
    return pl.pallas_call(
        _body,
        in_specs=[
            pl.BlockSpec(memory_space=pltpu.MemorySpace.SMEM),
            pl.BlockSpec(memory_space=pltpu.MemorySpace.VMEM),
            pl.BlockSpec(memory_space=pltpu.MemorySpace.HBM),
        ],
        out_specs=pl.BlockSpec(memory_space=pltpu.MemorySpace.HBM),
        out_shape=jax.ShapeDtypeStruct((_B, _S, _D), jnp.float32),
        scratch_shapes=[
            pltpu.VMEM((_B, 1, _W), jnp.float32),
            pltpu.VMEM((_B, 1, _W), jnp.float32),
            pltpu.SemaphoreType.DMA,
            pltpu.SemaphoreType.DMA,
        ],
    )(eot, W, hidden_states)


# speedup vs baseline: 47.0272x; 47.0272x over previous
import jax
import jax.numpy as jnp
from jax.experimental import pallas as pl
from jax.experimental.pallas import tpu as pltpu

_B, _S, _D = 4, 8192, 2048
_ST, _EN = 0, 1024
_W = _EN - _ST
_BS = 1024


def _body(eot_ref, w_ref, hid_blk_ref, hid_any_ref, out_ref, row_s, new_s, sem):
    b = pl.program_id(0)
    j = pl.program_id(1)

    out_ref[...] = hid_blk_ref[...]

    @pl.when(j == 0)
    def _gather_rotate():
        e = eot_ref[b]
        cp = pltpu.make_async_copy(
            hid_any_ref.at[pl.ds(b, 1), pl.ds(e, 1), pl.ds(_ST, _W)],
            row_s,
            sem,
        )
        cp.start()
        cp.wait()
        t = row_s[...].reshape(1, _W)
        r = jax.lax.dot_general(
            t, w_ref[...], (((1,), (0,)), ((), ())),
            preferred_element_type=jnp.float32,
        )
        inv = jax.lax.dot_general(
            r, w_ref[...], (((1,), (1,)), ((), ())),
            preferred_element_type=jnp.float32,
        )
        new_s[...] = inv.reshape(1, 1, _W)

    e = eot_ref[b]
    local = e - j * _BS

    @pl.when((local >= 0) & (local < _BS))
    def _patch():
        out_ref[pl.ds(0, 1), pl.ds(local, 1), pl.ds(_ST, _W)] = new_s[...]


def kernel(hidden_states, eot_indices, W):
    eot = eot_indices.astype(jnp.int32)
    return pl.pallas_call(
        _body,
        grid=(_B, _S // _BS),
        in_specs=[
            pl.BlockSpec(memory_space=pltpu.MemorySpace.SMEM),
            pl.BlockSpec((_W, _W), lambda b, j: (0, 0)),
            pl.BlockSpec((1, _BS, _D), lambda b, j: (b, j, 0)),
            pl.BlockSpec(memory_space=pltpu.MemorySpace.HBM),
        ],
        out_specs=pl.BlockSpec((1, _BS, _D), lambda b, j: (b, j, 0)),
        out_shape=jax.ShapeDtypeStruct((_B, _S, _D), jnp.float32),
        scratch_shapes=[
            pltpu.VMEM((1, 1, _W), jnp.float32),
            pltpu.VMEM((1, 1, _W), jnp.float32),
            pltpu.SemaphoreType.DMA,
        ],
        compiler_params=pltpu.CompilerParams(
            dimension_semantics=("parallel", "arbitrary"),
        ),
    )(eot, W, hidden_states, hidden_states)


# speedup vs baseline: 48.0826x; 1.0224x over previous
import jax
import jax.numpy as jnp
from jax.experimental import pallas as pl
from jax.experimental.pallas import tpu as pltpu

_B, _S, _D = 4, 8192, 2048
_ST, _EN = 0, 1024
_W = _EN - _ST
_BS = 1024


def _body(eot_ref, w_hbm_ref, hid_blk_ref, hid_any_ref, out_ref,
          w_s, rows_s, new_s, sem, wsem):
    b = pl.program_id(0)
    j = pl.program_id(1)

    out_ref[...] = hid_blk_ref[...]

    @pl.when((b == 0) & (j == 0))
    def _gather_rotate():
        wcp = pltpu.make_async_copy(w_hbm_ref, w_s, wsem)
        wcp.start()
        gathers = []
        for bb in range(_B):
            e = eot_ref[bb]
            cp = pltpu.make_async_copy(
                hid_any_ref.at[pl.ds(bb, 1), pl.ds(e, 1), pl.ds(_ST, _W)],
                rows_s.at[pl.ds(bb, 1)],
                sem,
            )
            cp.start()
            gathers.append(cp)
        for cp in gathers:
            cp.wait()
        wcp.wait()
        t = rows_s[...].reshape(_B, _W)
        r = jax.lax.dot_general(
            t, w_s[...], (((1,), (0,)), ((), ())),
            preferred_element_type=jnp.float32,
        )
        inv = jax.lax.dot_general(
            r, w_s[...], (((1,), (1,)), ((), ())),
            preferred_element_type=jnp.float32,
        )
        new_s[...] = inv.reshape(_B, 1, _W)

    e = eot_ref[b]
    local = e - j * _BS

    @pl.when((local >= 0) & (local < _BS))
    def _patch():
        out_ref[pl.ds(0, 1), pl.ds(local, 1), pl.ds(_ST, _W)] = (
            new_s[pl.ds(b, 1)]
        )


def kernel(hidden_states, eot_indices, W):
    eot = eot_indices.astype(jnp.int32)
    return pl.pallas_call(
        _body,
        grid=(_B, _S // _BS),
        in_specs=[
            pl.BlockSpec(memory_space=pltpu.MemorySpace.SMEM),
            pl.BlockSpec(memory_space=pltpu.MemorySpace.HBM),
            pl.BlockSpec((1, _BS, _D), lambda b, j: (b, j, 0)),
            pl.BlockSpec(memory_space=pltpu.MemorySpace.HBM),
        ],
        out_specs=pl.BlockSpec((1, _BS, _D), lambda b, j: (b, j, 0)),
        out_shape=jax.ShapeDtypeStruct((_B, _S, _D), jnp.float32),
        scratch_shapes=[
            pltpu.VMEM((_W, _W), jnp.float32),
            pltpu.VMEM((_B, 1, _W), jnp.float32),
            pltpu.VMEM((_B, 1, _W), jnp.float32),
            pltpu.SemaphoreType.DMA,
            pltpu.SemaphoreType.DMA,
        ],
        compiler_params=pltpu.CompilerParams(
            dimension_semantics=("arbitrary", "arbitrary"),
        ),
    )(eot, W, hidden_states, hidden_states)
